# baseline (speedup 1.0000x reference)
import functools

import jax
import jax.numpy as jnp
from jax import lax
from jax.experimental import pallas as pl
from jax.experimental.pallas import tpu as pltpu
from jax.experimental.pallas import tpu_sc as plsc

HIDDEN = 768
NV = HIDDEN // 16
EPS = 1e-12
CHUNK = 32


_GDN = lax.GatherDimensionNumbers(
    offset_dims=(), collapsed_slice_dims=(0,), start_index_map=(0,))


def _lane_shuffle(x, idx):
    return lax.gather(x, idx[:, None], _GDN, (1,),
                      mode=lax.GatherScatterMode.PROMISE_IN_BOUNDS)


def _allreduce_sum16(x):
    lanes = lax.iota(jnp.int32, 16)
    for sh in (8, 4, 2, 1):
        x = x + _lane_shuffle(x, lanes ^ sh)
    return x


def _rsqrt16(x):
    xi = lax.bitcast_convert_type(x, jnp.int32)
    yi = jnp.int32(0x5F3759DF) - (xi >> 1)
    y = lax.bitcast_convert_type(yi, jnp.float32)
    half = jnp.float32(0.5) * x
    for _ in range(3):
        y = y * (jnp.float32(1.5) - half * y * y)
    return y


def _sc_body(nw, ids_hbm, tids_hbm, tok_hbm, pos_hbm, type_hbm, gamma_hbm,
             beta_hbm, out_hbm, idx_v, tidx_v, rows_v, pos_v, trows_v,
             gamma_v, beta_v, sem_a, sem_b):
    info = plsc.get_sparse_core_info()
    nc = info.num_cores
    wid = lax.axis_index("s") * nc + lax.axis_index("c")
    tokens_per_w = (64 * 512) // nw
    chunks = tokens_per_w // CHUNK
    chunks_per_seq = 512 // CHUNK

    pltpu.sync_copy(gamma_hbm, gamma_v)
    pltpu.sync_copy(beta_hbm, beta_v)

    def chunk_body(ci, _):
        base = wid * tokens_per_w + ci * CHUNK
        s0 = (ci % chunks_per_seq) * CHUNK
        pltpu.sync_copy(ids_hbm.at[pl.ds(base, CHUNK)], idx_v)
        pltpu.sync_copy(tids_hbm.at[pl.ds(base, CHUNK)], tidx_v)
        pltpu.sync_copy(pos_hbm.at[pl.ds(s0, CHUNK)], pos_v)
        cp_a = pltpu.async_copy(tok_hbm.at[idx_v], rows_v, sem_a)
        cp_b = pltpu.async_copy(type_hbm.at[tidx_v], trows_v, sem_b)
        cp_a.wait()
        cp_b.wait()

        def token_body(t, _):
            def sum_body(j, carry):
                s, sq = carry
                d = pl.ds(j * 16, 16)
                v = rows_v[t, d] + pos_v[t, d] + trows_v[t, d]
                rows_v[t, d] = v
                return s + v, sq + v * v

            zero = jnp.zeros((16,), jnp.float32)
            s, sq = lax.fori_loop(0, NV, sum_body, (zero, zero))
            inv_n = jnp.float32(1.0 / HIDDEN)
            mean_v = _allreduce_sum16(s) * inv_n
            var_v = _allreduce_sum16(sq) * inv_n - mean_v * mean_v
            rstd = _rsqrt16(var_v + jnp.float32(EPS))

            def norm_body(j, _):
                d = pl.ds(j * 16, 16)
                v = rows_v[t, d]
                rows_v[t, d] = (v - mean_v) * rstd * gamma_v[d] + beta_v[d]
                return 0

            lax.fori_loop(0, NV, norm_body, 0)
            return 0

        lax.fori_loop(0, CHUNK, token_body, 0)
        pltpu.sync_copy(rows_v, out_hbm.at[pl.ds(base, CHUNK)])
        return 0

    lax.fori_loop(0, chunks, chunk_body, 0)


def kernel(input_ids, token_type_ids, token_table, pos_table, type_table,
           gamma, beta):
    bsz, seq = input_ids.shape
    mesh = plsc.VectorSubcoreMesh(core_axis_name="c", subcore_axis_name="s")
    nw = mesh.num_cores * mesh.num_subcores
    call = pl.kernel(
        functools.partial(_sc_body, nw),
        out_type=jax.ShapeDtypeStruct((bsz * seq, HIDDEN), jnp.float32),
        mesh=mesh,
        scratch_types=[
            pltpu.VMEM((CHUNK,), jnp.int32),
            pltpu.VMEM((CHUNK,), jnp.int32),
            pltpu.VMEM((CHUNK, HIDDEN), jnp.float32),
            pltpu.VMEM((CHUNK, HIDDEN), jnp.float32),
            pltpu.VMEM((CHUNK, HIDDEN), jnp.float32),
            pltpu.VMEM((HIDDEN,), jnp.float32),
            pltpu.VMEM((HIDDEN,), jnp.float32),
            pltpu.SemaphoreType.DMA,
            pltpu.SemaphoreType.DMA,
        ],
    )
    ids = input_ids.reshape(-1).astype(jnp.int32)
    tids = token_type_ids.reshape(-1).astype(jnp.int32)
    out = call(ids, tids, token_table, pos_table, type_table, gamma, beta)
    return out.reshape(bsz, seq, HIDDEN)


# speedup vs baseline: 1.8568x; 1.8568x over previous
import functools

import jax
import jax.numpy as jnp
from jax import lax
from jax.experimental import pallas as pl
from jax.experimental.pallas import tpu as pltpu
from jax.experimental.pallas import tpu_sc as plsc

HIDDEN = 768
NV = HIDDEN // 16
EPS = 1e-12
SLICE = 16
BB = 4
NBUF = 2

_GDN = lax.GatherDimensionNumbers(
    offset_dims=(), collapsed_slice_dims=(0,), start_index_map=(0,))


def _lane_shuffle(x, idx):
    return lax.gather(x, idx[:, None], _GDN, (1,),
                      mode=lax.GatherScatterMode.PROMISE_IN_BOUNDS)


def _allreduce_sum16(x):
    lanes = lax.iota(jnp.int32, 16)
    for sh in (8, 4, 2, 1):
        x = x + _lane_shuffle(x, lanes ^ sh)
    return x


def _rsqrt16(x):
    xi = lax.bitcast_convert_type(x, jnp.int32)
    yi = jnp.int32(0x5F3759DF) - (xi >> 1)
    y = lax.bitcast_convert_type(yi, jnp.float32)
    half = jnp.float32(0.5) * x
    for _ in range(3):
        y = y * (jnp.float32(1.5) - half * y * y)
    return y


def _sc_body(nbatch, seq_len, nw, ids_hbm, tids_hbm, tok_hbm, pos_hbm, type_hbm,
             gamma_hbm, beta_hbm, out_hbm, idx_v, tidx_v, rows_v, base_v,
             diff_v, type_v, gamma_v, beta_v, gsem0, gsem1, osem0, osem1):
    info = plsc.get_sparse_core_info()
    nc = info.num_cores
    wid = lax.axis_index("s") * nc + lax.axis_index("c")
    p0 = wid * SLICE
    nchunk = nbatch // BB
    lanes = lax.iota(jnp.int32, 16)

    pltpu.sync_copy(gamma_hbm, gamma_v)
    pltpu.sync_copy(beta_hbm, beta_v)
    pltpu.sync_copy(type_hbm, type_v)
    pltpu.sync_copy(pos_hbm.at[pl.ds(p0, SLICE)], base_v)

    def pre_body(i, _):
        r = i // NV
        j = i % NV
        d = pl.ds(j * 16, 16)
        base_v[r, d] = base_v[r, d] + type_v[0, d]
        return 0

    lax.fori_loop(0, SLICE * NV, pre_body, 0)

    def diff_body(j, _):
        d = pl.ds(j * 16, 16)
        diff_v[d] = type_v[1, d] - type_v[0, d]
        return 0

    lax.fori_loop(0, NV, diff_body, 0)

    gsems = (gsem0, gsem1)
    osems = (osem0, osem1)

    def fire_chunk(c, buf):
        b0 = c * BB
        for b in range(BB):
            off = (b0 + b) * seq_len + p0
            pltpu.sync_copy(ids_hbm.at[pl.ds(off, SLICE)], idx_v.at[buf, b])
            pltpu.sync_copy(tids_hbm.at[pl.ds(off, SLICE)],
                            tidx_v.at[buf, b])
        for b in range(BB):
            pltpu.async_copy(tok_hbm.at[idx_v.at[buf, b]],
                             rows_v.at[buf, b], gsems[buf])

    def drain_gathers(buf):
        for b in range(BB):
            pltpu.make_async_copy(tok_hbm.at[idx_v.at[buf, b]],
                                  rows_v.at[buf, b], gsems[buf]).wait()

    def fire_out(c, buf):
        b0 = c * BB
        for b in range(BB):
            pltpu.async_copy(rows_v.at[buf, b],
                             out_hbm.at[b0 + b, pl.ds(p0, SLICE)],
                             osems[buf])

    def drain_out(buf):
        for b in range(BB):
            pltpu.make_async_copy(rows_v.at[buf, b],
                                  out_hbm.at[b, pl.ds(p0, SLICE)],
                                  osems[buf]).wait()

    def compute_chunk(buf):
        for b in range(BB):
            tvec = tidx_v[buf, b, :]

            def token_body(sl, _):
                tsplat = _lane_shuffle(tvec, jnp.full((16,), sl, jnp.int32))
                tf = tsplat.astype(jnp.float32)
                zero = jnp.zeros((16,), jnp.float32)

                @plsc.parallel_loop(0, NV, unroll=8, carry=(zero, zero))
                def sums(j, carry):
                    s, sq = carry
                    d = pl.ds(j * 16, 16)
                    e = base_v[sl, d] + tf * diff_v[d]
                    v = rows_v[buf, b, sl, d] + e
                    rows_v[buf, b, sl, d] = v
                    return s + v, sq + v * v

                s, sq = sums
                inv_n = jnp.float32(1.0 / HIDDEN)
                mean_v = _allreduce_sum16(s) * inv_n
                var_v = _allreduce_sum16(sq) * inv_n - mean_v * mean_v
                rstd = _rsqrt16(var_v + jnp.float32(EPS))

                @plsc.parallel_loop(0, NV, unroll=8)
                def _(j):
                    d = pl.ds(j * 16, 16)
                    v = rows_v[buf, b, sl, d]
                    rows_v[buf, b, sl, d] = (
                        (v - mean_v) * rstd * gamma_v[d] + beta_v[d])

                return 0

            lax.fori_loop(0, SLICE, token_body, 0)

    pairs = nchunk // 2
    fire_chunk(0, 0)

    def pair_body(i, _):
        c0 = 2 * i

        @pl.when(i > 0)
        def _():
            drain_out(1)

        fire_chunk(c0 + 1, 1)
        drain_gathers(0)
        compute_chunk(0)
        fire_out(c0, 0)

        @pl.when(i < pairs - 1)
        def _():
            drain_out(0)
            fire_chunk(c0 + 2, 0)

        drain_gathers(1)
        compute_chunk(1)
        fire_out(c0 + 1, 1)
        return 0

    lax.fori_loop(0, pairs, pair_body, 0)
    drain_out(0)
    drain_out(1)


def kernel(input_ids, token_type_ids, token_table, pos_table, type_table,
           gamma, beta):
    bsz, seq = input_ids.shape
    mesh = plsc.VectorSubcoreMesh(core_axis_name="c", subcore_axis_name="s")
    nw = mesh.num_cores * mesh.num_subcores
    call = pl.kernel(
        functools.partial(_sc_body, bsz, seq, nw),
        out_type=jax.ShapeDtypeStruct((bsz, seq, HIDDEN), jnp.float32),
        mesh=mesh,
        scratch_types=[
            pltpu.VMEM((NBUF, BB, SLICE), jnp.int32),
            pltpu.VMEM((NBUF, BB, SLICE), jnp.int32),
            pltpu.VMEM((NBUF, BB, SLICE, HIDDEN), jnp.float32),
            pltpu.VMEM((SLICE, HIDDEN), jnp.float32),
            pltpu.VMEM((HIDDEN,), jnp.float32),
            pltpu.VMEM((2, HIDDEN), jnp.float32),
            pltpu.VMEM((HIDDEN,), jnp.float32),
            pltpu.VMEM((HIDDEN,), jnp.float32),
            pltpu.SemaphoreType.DMA,
            pltpu.SemaphoreType.DMA,
            pltpu.SemaphoreType.DMA,
            pltpu.SemaphoreType.DMA,
        ],
    )
    ids = input_ids.reshape(-1).astype(jnp.int32)
    tids = token_type_ids.reshape(-1).astype(jnp.int32)
    return call(ids, tids, token_table, pos_table, type_table, gamma, beta)


# speedup vs baseline: 5.8316x; 3.1407x over previous
import functools

import jax
import jax.numpy as jnp
from jax import lax
from jax.experimental import pallas as pl
from jax.experimental.pallas import tpu as pltpu
from jax.experimental.pallas import tpu_sc as plsc

HIDDEN = 768
EPS = 1e-12
GCHUNK = 64


def _sc_gather_body(ntok, ids_hbm, tok_hbm, out_hbm, idx_v, rows_v, gsem,
                    osem):
    info = plsc.get_sparse_core_info()
    nc = info.num_cores
    wid = lax.axis_index("s") * nc + lax.axis_index("c")
    nw = nc * info.num_subcores
    per_w = ntok // nw
    nchunk = per_w // GCHUNK
    base = wid * per_w
    pltpu.sync_copy(ids_hbm.at[pl.ds(base, per_w)], idx_v)

    def fire_gather(c, buf):
        return pltpu.async_copy(tok_hbm.at[idx_v.at[pl.ds(c * GCHUNK, GCHUNK)]],
                                rows_v.at[buf], gsem)

    def fire_out(c, buf):
        return pltpu.async_copy(
            rows_v.at[buf], out_hbm.at[pl.ds(base + c * GCHUNK, GCHUNK)],
            osem)

    def drain(sem, buf):
        pltpu.make_async_copy(rows_v.at[buf],
                              out_hbm.at[pl.ds(base, GCHUNK)], sem).wait()

    fire_gather(0, 0)

    def pair_body(i, _):
        c0 = 2 * i

        @pl.when(i > 0)
        def _():
            drain(osem, 1)

        drain(gsem, 0)
        fire_gather(c0 + 1, 1)
        fire_out(c0, 0)
        drain(gsem, 1)

        @pl.when(i < nchunk // 2 - 1)
        def _():
            drain(osem, 0)
            fire_gather(c0 + 2, 0)

        fire_out(c0 + 1, 1)
        return 0

    lax.fori_loop(0, nchunk // 2, pair_body, 0)
    drain(osem, 0)
    drain(osem, 1)


def _tc_ln_body(gath_ref, tids_ref, pos_ref, type_ref, gamma_ref, beta_ref,
                o_ref):
    x = gath_ref[0]
    tf = tids_ref[0].astype(jnp.float32)
    pos = pos_ref[...]
    t0 = type_ref[0, :][None, :]
    dlt = (type_ref[1, :] - type_ref[0, :])[None, :]
    x = x + pos + t0 + tf.T * dlt
    mean = jnp.mean(x, axis=-1, keepdims=True)
    xc = x - mean
    var = jnp.mean(xc * xc, axis=-1, keepdims=True)
    normed = xc * lax.rsqrt(var + EPS)
    o_ref[0] = normed * gamma_ref[...][None, :] + beta_ref[...][None, :]


def kernel(input_ids, token_type_ids, token_table, pos_table, type_table,
           gamma, beta):
    bsz, seq = input_ids.shape
    ntok = bsz * seq
    ids = input_ids.reshape(-1).astype(jnp.int32)

    mesh = plsc.VectorSubcoreMesh(core_axis_name="c", subcore_axis_name="s")
    nw = mesh.num_cores * mesh.num_subcores
    per_w = ntok // nw
    gather_call = pl.kernel(
        functools.partial(_sc_gather_body, ntok),
        out_type=jax.ShapeDtypeStruct((ntok, HIDDEN), jnp.float32),
        mesh=mesh,
        scratch_types=[
            pltpu.VMEM((per_w,), jnp.int32),
            pltpu.VMEM((2, GCHUNK, HIDDEN), jnp.float32),
            pltpu.SemaphoreType.DMA,
            pltpu.SemaphoreType.DMA,
        ],
    )
    gathered = gather_call(ids, token_table)

    tids = token_type_ids.reshape(bsz, 1, seq).astype(jnp.int32)
    out = pl.pallas_call(
        _tc_ln_body,
        grid=(bsz,),
        in_specs=[
            pl.BlockSpec((1, seq, HIDDEN), lambda i: (i, 0, 0)),
            pl.BlockSpec((1, 1, seq), lambda i: (i, 0, 0)),
            pl.BlockSpec((seq, HIDDEN), lambda i: (0, 0)),
            pl.BlockSpec((2, HIDDEN), lambda i: (0, 0)),
            pl.BlockSpec((HIDDEN,), lambda i: (0,)),
            pl.BlockSpec((HIDDEN,), lambda i: (0,)),
        ],
        out_specs=pl.BlockSpec((1, seq, HIDDEN), lambda i: (i, 0, 0)),
        out_shape=jax.ShapeDtypeStruct((bsz, seq, HIDDEN), jnp.float32),
    )(gathered.reshape(bsz, seq, HIDDEN), tids, pos_table, type_table,
      gamma, beta)
    return out


# speedup vs baseline: 6.1749x; 1.0589x over previous
import functools

import jax
import jax.numpy as jnp
from jax import lax
from jax.experimental import pallas as pl
from jax.experimental.pallas import tpu as pltpu
from jax.experimental.pallas import tpu_sc as plsc

HIDDEN = 768
EPS = 1e-12
GCHUNK = 64
NSPLIT = 4


def _sc_gather_body(ntok, ids_hbm, tok_hbm, out_hbm, idx_v, rows_v, gsem,
                    osem):
    info = plsc.get_sparse_core_info()
    nc = info.num_cores
    wid = lax.axis_index("s") * nc + lax.axis_index("c")
    nw = nc * info.num_subcores
    per_w = ntok // nw
    nchunk = per_w // GCHUNK
    base = wid * per_w
    pltpu.sync_copy(ids_hbm.at[pl.ds(base, per_w)], idx_v)

    def fire_gather(c, buf):
        return pltpu.async_copy(
            tok_hbm.at[idx_v.at[pl.ds(c * GCHUNK, GCHUNK)]], rows_v.at[buf],
            gsem)

    def fire_out(c, buf):
        return pltpu.async_copy(
            rows_v.at[buf], out_hbm.at[pl.ds(base + c * GCHUNK, GCHUNK)],
            osem)

    def drain(sem, buf):
        pltpu.make_async_copy(rows_v.at[buf],
                              out_hbm.at[pl.ds(base, GCHUNK)], sem).wait()

    fire_gather(0, 0)

    def pair_body(i, _):
        c0 = 2 * i

        @pl.when(i > 0)
        def _():
            drain(osem, 1)

        drain(gsem, 0)
        fire_gather(c0 + 1, 1)
        fire_out(c0, 0)
        drain(gsem, 1)

        @pl.when(i < nchunk // 2 - 1)
        def _():
            drain(osem, 0)
            fire_gather(c0 + 2, 0)

        fire_out(c0 + 1, 1)
        return 0

    lax.fori_loop(0, nchunk // 2, pair_body, 0)
    drain(osem, 0)
    drain(osem, 1)


def _ln_math(gath_ref, tids_ref, pos_ref, type_ref, gamma_ref, beta_ref,
             o_ref):
    x = gath_ref[0]
    tf = tids_ref[0].astype(jnp.float32)
    pos = pos_ref[...]
    t0 = type_ref[0, :][None, :]
    dlt = (type_ref[1, :] - type_ref[0, :])[None, :]
    x = x + pos + t0 + tf.T * dlt
    mean = jnp.mean(x, axis=-1, keepdims=True)
    xc = x - mean
    var = jnp.mean(xc * xc, axis=-1, keepdims=True)
    normed = xc * lax.rsqrt(var + EPS)
    o_ref[0] = normed * gamma_ref[...][None, :] + beta_ref[...][None, :]


def _tc_ln_chain(prev_ref, gath_ref, tids_ref, pos_ref, type_ref, gamma_ref,
                 beta_ref, o_ref):
    del prev_ref
    _ln_math(gath_ref, tids_ref, pos_ref, type_ref, gamma_ref, beta_ref,
             o_ref)


def kernel(input_ids, token_type_ids, token_table, pos_table, type_table,
           gamma, beta):
    bsz, seq = input_ids.shape
    ntok = bsz * seq
    ids = input_ids.reshape(-1).astype(jnp.int32)
    tids = token_type_ids.reshape(bsz, 1, seq).astype(jnp.int32)

    mesh = plsc.VectorSubcoreMesh(core_axis_name="c", subcore_axis_name="s")
    bq = bsz // NSPLIT
    tq = ntok // NSPLIT

    gather_call = pl.kernel(
        functools.partial(_sc_gather_body, tq),
        out_type=jax.ShapeDtypeStruct((tq, HIDDEN), jnp.float32),
        mesh=mesh,
        scratch_types=[
            pltpu.VMEM((tq // (mesh.num_cores * mesh.num_subcores),),
                       jnp.int32),
            pltpu.VMEM((2, GCHUNK, HIDDEN), jnp.float32),
            pltpu.SemaphoreType.DMA,
            pltpu.SemaphoreType.DMA,
        ],
    )
    gathered = [gather_call(ids[k * tq:(k + 1) * tq], token_table)
                for k in range(NSPLIT)]

    common_specs = [
        pl.BlockSpec((1, 1, seq), lambda i: (i, 0, 0)),
        pl.BlockSpec((seq, HIDDEN), lambda i: (0, 0)),
        pl.BlockSpec((2, HIDDEN), lambda i: (0, 0)),
        pl.BlockSpec((HIDDEN,), lambda i: (0,)),
        pl.BlockSpec((HIDDEN,), lambda i: (0,)),
    ]
    out_shape = jax.ShapeDtypeStruct((bsz, seq, HIDDEN), jnp.float32)

    out = None
    for k in range(NSPLIT):
        gk = gathered[k].reshape(bq, seq, HIDDEN)
        tk = tids[k * bq:(k + 1) * bq]
        om = functools.partial(lambda kk, i: (kk * bq + i, 0, 0), k)
        if out is None:
            out = pl.pallas_call(
                _ln_math,
                grid=(bq,),
                in_specs=[pl.BlockSpec((1, seq, HIDDEN),
                                       lambda i: (i, 0, 0))] + common_specs,
                out_specs=pl.BlockSpec((1, seq, HIDDEN), om),
                out_shape=out_shape,
            )(gk, tk, pos_table, type_table, gamma, beta)
        else:
            out = pl.pallas_call(
                _tc_ln_chain,
                grid=(bq,),
                in_specs=[
                    pl.BlockSpec(memory_space=pl.ANY),
                    pl.BlockSpec((1, seq, HIDDEN), lambda i: (i, 0, 0)),
                ] + common_specs,
                out_specs=pl.BlockSpec((1, seq, HIDDEN), om),
                out_shape=out_shape,
                input_output_aliases={0: 0},
            )(out, gk, tk, pos_table, type_table, gamma, beta)
    return out
